# initial kernel scaffold (unmeasured)
import jax
import jax.numpy as jnp
from jax import lax
from jax.experimental import pallas as pl
from jax.experimental.pallas import tpu as pltpu

N_DEV = 32


def kernel(x, router_W, route_idx, expert_W):
    n_tok, d = x.shape
    e_per, _, h = expert_W.shape
    n_exp = N_DEV * e_per
    n_hops = N_DEV - 1

    def body(x_ref, rw_ref, idx_ref, ew_ref, out_ref, gath_ref,
             send_sems, recv_sems):
        my = lax.axis_index("i")
        left = lax.rem(my - 1 + N_DEV, N_DEV)
        right = lax.rem(my + 1, N_DEV)

        barrier_sem = pltpu.get_barrier_semaphore()
        for nbr in (left, right):
            pl.semaphore_signal(
                barrier_sem, inc=1,
                device_id=(nbr,), device_id_type=pl.DeviceIdType.MESH,
            )
        pl.semaphore_wait(barrier_sem, 2)

        xv = x_ref[...]
        scores = jnp.dot(xv, rw_ref[...], preferred_element_type=jnp.float32)
        m = jnp.max(scores, axis=-1, keepdims=True)
        p = jnp.exp(scores - m)
        probs = p / jnp.sum(p, axis=-1, keepdims=True)
        e0 = idx_ref[:, 0:1]
        e1 = idx_ref[:, 1:2]
        eid = lax.broadcasted_iota(jnp.int32, (n_tok, n_exp), 1)
        one0 = (eid == e0).astype(jnp.float32)
        one1 = (eid == e1).astype(jnp.float32)
        g0 = jnp.sum(probs * one0, axis=-1, keepdims=True)
        g1 = jnp.sum(probs * one1, axis=-1, keepdims=True)
        gs = g0 + g1
        w0 = g0 / gs
        w1 = g1 / gs

        gath_ref[pl.ds(my * e_per, e_per)] = ew_ref[...]

        def contrib(origin, acc):
            wpair = gath_ref[pl.ds(origin * e_per, e_per)]
            for k in range(e_per):
                e = origin * e_per + k
                ce = (jnp.where(e0 == e, w0, 0.0)
                      + jnp.where(e1 == e, w1, 0.0))
                acc = acc + ce * jnp.dot(
                    xv, wpair[k], preferred_element_type=jnp.float32)
            return acc

        acc = contrib(my, jnp.zeros((n_tok, h), jnp.float32))

        for t in range(n_hops):
            src_slot = lax.rem(my - t + N_DEV, N_DEV)
            rdma = pltpu.make_async_remote_copy(
                src_ref=gath_ref.at[pl.ds(src_slot * e_per, e_per)],
                dst_ref=gath_ref.at[pl.ds(src_slot * e_per, e_per)],
                send_sem=send_sems.at[t],
                recv_sem=recv_sems.at[t],
                device_id=(right,),
                device_id_type=pl.DeviceIdType.MESH,
            )
            rdma.start()
            rdma.wait()
            origin = lax.rem(my - t - 1 + N_DEV, N_DEV)
            acc = contrib(origin, acc)

        out_ref[...] = acc

    return pl.pallas_call(
        body,
        out_shape=jax.ShapeDtypeStruct((n_tok, h), jnp.float32),
        in_specs=[
            pl.BlockSpec(memory_space=pltpu.VMEM),
            pl.BlockSpec(memory_space=pltpu.VMEM),
            pl.BlockSpec(memory_space=pltpu.VMEM),
            pl.BlockSpec(memory_space=pltpu.VMEM),
        ],
        out_specs=pl.BlockSpec(memory_space=pltpu.VMEM),
        scratch_shapes=[
            pltpu.VMEM((n_exp, d, h), jnp.float32),
            pltpu.SemaphoreType.DMA((n_hops,)),
            pltpu.SemaphoreType.DMA((n_hops,)),
        ],
        compiler_params=pltpu.CompilerParams(collective_id=0),
    )(x, router_W, route_idx, expert_W)


# baseline (device time: 432920 ns/iter reference)
import jax
import jax.numpy as jnp
from jax import lax
from jax.experimental import pallas as pl
from jax.experimental.pallas import tpu as pltpu

N_DEV = 32


def kernel(x, router_W, route_idx, expert_W):
    n_tok, d = x.shape
    e_per, _, h = expert_W.shape
    n_exp = N_DEV * e_per
    n_hops = N_DEV - 1

    def body(x_ref, rw_ref, idx_ref, ew_ref, out_ref, gath_ref,
             send_sems, recv_sems):
        my = lax.axis_index("i")
        left = lax.rem(my - 1 + N_DEV, N_DEV)
        right = lax.rem(my + 1, N_DEV)

        barrier_sem = pltpu.get_barrier_semaphore()
        for nbr in (left, right):
            pl.semaphore_signal(
                barrier_sem, inc=1,
                device_id=(nbr,), device_id_type=pl.DeviceIdType.MESH,
            )
        pl.semaphore_wait(barrier_sem, 2)

        xv = x_ref[...]
        scores = jnp.dot(xv, rw_ref[...], preferred_element_type=jnp.float32)
        m = jnp.max(scores, axis=-1, keepdims=True)
        p = jnp.exp(scores - m)
        probs = p / jnp.sum(p, axis=-1, keepdims=True)
        e0 = idx_ref[:, 0:1]
        e1 = idx_ref[:, 1:2]
        eid = lax.broadcasted_iota(jnp.int32, (n_tok, n_exp), 1)
        one0 = (eid == e0).astype(jnp.float32)
        one1 = (eid == e1).astype(jnp.float32)
        g0 = jnp.sum(probs * one0, axis=-1, keepdims=True)
        g1 = jnp.sum(probs * one1, axis=-1, keepdims=True)
        gs = g0 + g1
        w0 = g0 / gs
        w1 = g1 / gs

        gath_ref[pl.ds(my * e_per, e_per)] = ew_ref[...]

        def contrib(origin, acc):
            wpair = gath_ref[pl.ds(origin * e_per, e_per)]
            for k in range(e_per):
                e = origin * e_per + k
                ce = (jnp.where(e0 == e, w0, 0.0)
                      + jnp.where(e1 == e, w1, 0.0))
                acc = acc + ce * jnp.dot(
                    xv, wpair[k], preferred_element_type=jnp.float32)
            return acc

        acc = contrib(my, jnp.zeros((n_tok, h), jnp.float32))

        for t in range(n_hops):
            src_slot = lax.rem(my - t + N_DEV, N_DEV)
            rdma = pltpu.make_async_remote_copy(
                src_ref=gath_ref.at[pl.ds(src_slot * e_per, e_per)],
                dst_ref=gath_ref.at[pl.ds(src_slot * e_per, e_per)],
                send_sem=send_sems.at[t],
                recv_sem=recv_sems.at[t],
                device_id=(right,),
                device_id_type=pl.DeviceIdType.MESH,
            )
            rdma.start()
            rdma.wait()
            origin = lax.rem(my - t - 1 + N_DEV, N_DEV)
            acc = contrib(origin, acc)

        out_ref[...] = acc

    return pl.pallas_call(
        body,
        out_shape=jax.ShapeDtypeStruct((n_tok, h), jnp.float32),
        in_specs=[
            pl.BlockSpec(memory_space=pltpu.VMEM),
            pl.BlockSpec(memory_space=pltpu.VMEM),
            pl.BlockSpec(memory_space=pltpu.VMEM),
            pl.BlockSpec(memory_space=pltpu.VMEM),
        ],
        out_specs=pl.BlockSpec(memory_space=pltpu.VMEM),
        scratch_shapes=[
            pltpu.VMEM((n_exp, d, h), jnp.float32),
            pltpu.SemaphoreType.DMA((n_hops,)),
            pltpu.SemaphoreType.DMA((n_hops,)),
        ],
        compiler_params=pltpu.CompilerParams(
            collective_id=0,
            vmem_limit_bytes=100 * 1024 * 1024,
        ),
    )(x, router_W, route_idx, expert_W)


# device time: 232862 ns/iter; 1.8591x vs baseline; 1.8591x over previous
import jax
import jax.numpy as jnp
from jax import lax
from jax.experimental import pallas as pl
from jax.experimental.pallas import tpu as pltpu

N_DEV = 32


def kernel(x, router_W, route_idx, expert_W):
    n_tok, d = x.shape
    e_per, _, h = expert_W.shape
    n_exp = N_DEV * e_per
    n_hops = N_DEV - 1

    assert e_per == 2, "bidirectional split assumes 2 experts per device"

    def body(x_ref, rw_ref, idx_ref, ew_ref, out_ref, gath_ref,
             r_send, r_recv, l_send, l_recv):
        my = lax.axis_index("i")
        left = lax.rem(my - 1 + N_DEV, N_DEV)
        right = lax.rem(my + 1, N_DEV)

        barrier_sem = pltpu.get_barrier_semaphore()
        for nbr in (left, right):
            pl.semaphore_signal(
                barrier_sem, inc=1,
                device_id=(nbr,), device_id_type=pl.DeviceIdType.MESH,
            )
        pl.semaphore_wait(barrier_sem, 2)

        xv32 = x_ref[...]
        xv = xv32.astype(jnp.bfloat16)
        scores = jnp.dot(xv32, rw_ref[...], preferred_element_type=jnp.float32)
        m = jnp.max(scores, axis=-1, keepdims=True)
        p = jnp.exp(scores - m)
        probs = p / jnp.sum(p, axis=-1, keepdims=True)
        e0 = idx_ref[:, 0:1]
        e1 = idx_ref[:, 1:2]
        eid = lax.broadcasted_iota(jnp.int32, (n_tok, n_exp), 1)
        one0 = (eid == e0).astype(jnp.float32)
        one1 = (eid == e1).astype(jnp.float32)
        g0 = jnp.sum(probs * one0, axis=-1, keepdims=True)
        g1 = jnp.sum(probs * one1, axis=-1, keepdims=True)
        gs = g0 + g1
        w0 = g0 / gs
        w1 = g1 / gs

        gath_ref[pl.ds(my * e_per, e_per)] = ew_ref[...].astype(jnp.bfloat16)

        def contrib(e, acc):
            w = gath_ref[pl.ds(e, 1)][0]
            ce = (jnp.where(e0 == e, w0, 0.0)
                  + jnp.where(e1 == e, w1, 0.0))
            return acc + ce * jnp.dot(
                xv, w, preferred_element_type=jnp.float32)

        def desc(t, row, dst, send_sems, recv_sems):
            return pltpu.make_async_remote_copy(
                src_ref=gath_ref.at[pl.ds(row, 1)],
                dst_ref=gath_ref.at[pl.ds(row, 1)],
                send_sem=send_sems.at[t],
                recv_sem=recv_sems.at[t],
                device_id=(dst,),
                device_id_type=pl.DeviceIdType.MESH,
            )

        rd = desc(0, my * 2, right, r_send, r_recv)
        rd.start()
        ld = desc(0, my * 2 + 1, left, l_send, l_recv)
        ld.start()
        rds, lds = {0: rd}, {0: ld}

        acc = contrib(my * 2, jnp.zeros((n_tok, h), jnp.float32))
        acc = contrib(my * 2 + 1, acc)

        for t in range(n_hops):
            rds[t].wait()
            lds[t].wait()
            e_cw = lax.rem((my - t - 1 + N_DEV) * 2, n_exp)
            e_ccw = lax.rem((my + t + 1) * 2, n_exp) + 1
            if t + 1 < n_hops:
                rd = desc(t + 1, e_cw, right, r_send, r_recv)
                rd.start()
                ld = desc(t + 1, e_ccw, left, l_send, l_recv)
                ld.start()
                rds[t + 1], lds[t + 1] = rd, ld
            acc = contrib(e_cw, acc)
            acc = contrib(e_ccw, acc)

        out_ref[...] = acc

    return pl.pallas_call(
        body,
        out_shape=jax.ShapeDtypeStruct((n_tok, h), jnp.float32),
        in_specs=[
            pl.BlockSpec(memory_space=pltpu.VMEM),
            pl.BlockSpec(memory_space=pltpu.VMEM),
            pl.BlockSpec(memory_space=pltpu.VMEM),
            pl.BlockSpec(memory_space=pltpu.VMEM),
        ],
        out_specs=pl.BlockSpec(memory_space=pltpu.VMEM),
        scratch_shapes=[
            pltpu.VMEM((n_exp, d, h), jnp.bfloat16),
            pltpu.SemaphoreType.DMA((n_hops,)),
            pltpu.SemaphoreType.DMA((n_hops,)),
            pltpu.SemaphoreType.DMA((n_hops,)),
            pltpu.SemaphoreType.DMA((n_hops,)),
        ],
        compiler_params=pltpu.CompilerParams(
            collective_id=0,
            vmem_limit_bytes=100 * 1024 * 1024,
        ),
    )(x, router_W, route_idx, expert_W)


# device time: 193827 ns/iter; 2.2335x vs baseline; 1.2014x over previous
import jax
import jax.numpy as jnp
from jax import lax
from jax.experimental import pallas as pl
from jax.experimental.pallas import tpu as pltpu

N_DEV = 32


def kernel(x, router_W, route_idx, expert_W):
    n_tok, d = x.shape
    e_per, _, h = expert_W.shape
    n_exp = N_DEV * e_per
    n_cw = N_DEV // 2
    n_ccw = N_DEV // 2 - 1

    assert e_per == 2

    def body(x_ref, rw_ref, idx_ref, ew_ref, out_ref, gath_ref,
             r_send, r_recv, l_send, l_recv):
        my = lax.axis_index("i")
        left = lax.rem(my - 1 + N_DEV, N_DEV)
        right = lax.rem(my + 1, N_DEV)

        barrier_sem = pltpu.get_barrier_semaphore()
        for nbr in (left, right):
            pl.semaphore_signal(
                barrier_sem, inc=1,
                device_id=(nbr,), device_id_type=pl.DeviceIdType.MESH,
            )
        pl.semaphore_wait(barrier_sem, 2)

        xv32 = x_ref[...]
        xv = xv32.astype(jnp.bfloat16)
        scores = jnp.dot(xv32, rw_ref[...], preferred_element_type=jnp.float32)
        m = jnp.max(scores, axis=-1, keepdims=True)
        p = jnp.exp(scores - m)
        probs = p / jnp.sum(p, axis=-1, keepdims=True)
        e0 = idx_ref[:, 0:1]
        e1 = idx_ref[:, 1:2]
        eid = lax.broadcasted_iota(jnp.int32, (n_tok, n_exp), 1)
        one0 = (eid == e0).astype(jnp.float32)
        one1 = (eid == e1).astype(jnp.float32)
        g0 = jnp.sum(probs * one0, axis=-1, keepdims=True)
        g1 = jnp.sum(probs * one1, axis=-1, keepdims=True)
        gs = g0 + g1
        w0 = g0 / gs
        w1 = g1 / gs

        gath_ref[pl.ds(my * e_per, e_per)] = ew_ref[...].astype(jnp.bfloat16)

        def contrib(origin, acc):
            wpair = gath_ref[pl.ds(origin * e_per, e_per)]
            for k in range(e_per):
                e = origin * e_per + k
                ce = (jnp.where(e0 == e, w0, 0.0)
                      + jnp.where(e1 == e, w1, 0.0))
                acc = acc + ce * jnp.dot(
                    xv, wpair[k], preferred_element_type=jnp.float32)
            return acc

        def desc(t, origin, dst, send_sems, recv_sems):
            return pltpu.make_async_remote_copy(
                src_ref=gath_ref.at[pl.ds(origin * e_per, e_per)],
                dst_ref=gath_ref.at[pl.ds(origin * e_per, e_per)],
                send_sem=send_sems.at[t],
                recv_sem=recv_sems.at[t],
                device_id=(dst,),
                device_id_type=pl.DeviceIdType.MESH,
            )

        rds = {0: desc(0, my, right, r_send, r_recv)}
        rds[0].start()
        lds = {0: desc(0, my, left, l_send, l_recv)}
        lds[0].start()

        acc = contrib(my, jnp.zeros((n_tok, h), jnp.float32))

        for t in range(n_cw):
            o_cw = lax.rem(my - t - 1 + N_DEV, N_DEV)
            rds[t].wait_recv()
            if t + 1 < n_cw:
                rds[t + 1] = desc(t + 1, o_cw, right, r_send, r_recv)
                rds[t + 1].start()
            if t < n_ccw:
                o_ccw = lax.rem(my + t + 1, N_DEV)
                lds[t].wait_recv()
                if t + 1 < n_ccw:
                    lds[t + 1] = desc(t + 1, o_ccw, left, l_send, l_recv)
                    lds[t + 1].start()
                acc = contrib(o_ccw, acc)
            acc = contrib(o_cw, acc)

        out_ref[...] = acc

        for t in range(n_cw):
            rds[t].wait_send()
        for t in range(n_ccw):
            lds[t].wait_send()

    return pl.pallas_call(
        body,
        out_shape=jax.ShapeDtypeStruct((n_tok, h), jnp.float32),
        in_specs=[
            pl.BlockSpec(memory_space=pltpu.VMEM),
            pl.BlockSpec(memory_space=pltpu.VMEM),
            pl.BlockSpec(memory_space=pltpu.VMEM),
            pl.BlockSpec(memory_space=pltpu.VMEM),
        ],
        out_specs=pl.BlockSpec(memory_space=pltpu.VMEM),
        scratch_shapes=[
            pltpu.VMEM((n_exp, d, h), jnp.bfloat16),
            pltpu.SemaphoreType.DMA((n_cw,)),
            pltpu.SemaphoreType.DMA((n_cw,)),
            pltpu.SemaphoreType.DMA((n_ccw,)),
            pltpu.SemaphoreType.DMA((n_ccw,)),
        ],
        compiler_params=pltpu.CompilerParams(
            collective_id=0,
            vmem_limit_bytes=100 * 1024 * 1024,
        ),
    )(x, router_W, route_idx, expert_W)
